# baseline (speedup 1.0000x reference)
import jax
import jax.numpy as jnp
from jax.experimental import pallas as pl
from jax.experimental.pallas import tpu as pltpu

_C = 384
_MUL = 1.0 / 40.0
_T_MAX = 2.0
_BX = 4


def _body(a_ref, b_ref, out_ref):
    a = a_ref[0]
    b = b_ref[0]
    d = ((a[..., :_C] - a[..., _C:]) + (b[..., :_C] - b[..., _C:])) * _MUL
    out_ref[:, :, :_C] = jnp.minimum(d + _T_MAX, _T_MAX)
    out_ref[:, :, _C:] = jnp.minimum(_T_MAX - d, _T_MAX)


def kernel(tj1, tj2):
    t1 = jnp.transpose(tj1, (0, 2, 3, 1))
    t2 = jnp.transpose(tj2, (0, 2, 3, 1))
    out = pl.pallas_call(
        _body,
        grid=(64 // _BX,),
        in_specs=[pl.BlockSpec((1, _BX, 64, 2 * _C), lambda i: (0, i, 0, 0)),
                  pl.BlockSpec((1, _BX, 64, 2 * _C), lambda i: (0, i, 0, 0))],
        out_specs=pl.BlockSpec((_BX, 64, 2 * _C), lambda i: (i, 0, 0)),
        out_shape=jax.ShapeDtypeStruct((64, 64, 2 * _C), jnp.float32),
    )(t1, t2)
    return jnp.transpose(out, (2, 0, 1))


# speedup vs baseline: 1.3507x; 1.3507x over previous
import jax
import jax.numpy as jnp
from jax.experimental import pallas as pl
from jax.experimental.pallas import tpu as pltpu

_C = 384
_MUL = 1.0 / 40.0
_T_MAX = 2.0
_BX = 16


def _body(a_ref, b_ref, out_ref):
    a = a_ref[0]
    b = b_ref[0]
    d = ((a[..., :_C] - a[..., _C:]) + (b[..., :_C] - b[..., _C:])) * _MUL
    out_ref[:, :, :_C] = jnp.minimum(d + _T_MAX, _T_MAX)
    out_ref[:, :, _C:] = jnp.minimum(_T_MAX - d, _T_MAX)


def kernel(tj1, tj2):
    t1 = jnp.transpose(tj1, (0, 2, 3, 1))
    t2 = jnp.transpose(tj2, (0, 2, 3, 1))
    out = pl.pallas_call(
        _body,
        grid=(64 // _BX,),
        in_specs=[pl.BlockSpec((1, _BX, 64, 2 * _C), lambda i: (0, i, 0, 0)),
                  pl.BlockSpec((1, _BX, 64, 2 * _C), lambda i: (0, i, 0, 0))],
        out_specs=pl.BlockSpec((_BX, 64, 2 * _C), lambda i: (i, 0, 0)),
        out_shape=jax.ShapeDtypeStruct((64, 64, 2 * _C), jnp.float32),
    )(t1, t2)
    return jnp.transpose(out, (2, 0, 1))
